# Initial kernel scaffold; baseline (speedup 1.0000x reference)
import jax
import jax.numpy as jnp
from jax.experimental import pallas as pl


def kernel(labels, edge_index, mask):
    raise NotImplementedError("write your pallas kernel here")


# speedup vs baseline: 2.7361x; 2.7361x over previous
import jax
import jax.numpy as jnp
from jax import lax
from jax.experimental import pallas as pl
from jax.experimental.pallas import tpu as pltpu
from jax.experimental.pallas import tpu_sc as plsc

N = 100000
C = 40
HQ = 8
NPL = 5
E = 1600000
ALPHA = 0.9

B = 128
RPW = 784
ROWS2 = 16 * RPW
EPAD = ROWS2 * B
SCB = 8
SCH = RPW // SCB

NPW = 6256
NPAD = 16 * NPW
RC = 368
RC_TAIL = 272

NACC = NPAD
EW_ROWS = 400
EW_CHUNKS = N // EW_ROWS

_F32 = jnp.float32
_I32 = jnp.int32


def _rsqrt16(d):
    i = lax.bitcast_convert_type(d, _I32)
    i = jnp.int32(0x5F3759DF) - (i >> 1)
    x = lax.bitcast_convert_type(i, _F32)
    for _ in range(3):
        x = x * (1.5 - 0.5 * d * x * x)
    return x


def _prep_body(dst2, labelsS, maskf, zeros1,
               zS, R2S, R1S, A2S, A1S,
               degs, ibuf, ones128, dbufn, mbuf, nbuf,
               lbuf, zb, r2b, r1b, a2b, a1b):
    w = lax.axis_index("s")
    k = lax.axis_index("c")
    iota = lax.iota(_I32, 16)
    npass = jnp.where(k == 0, 3, 2)

    for i in range(8):
        ones128[pl.ds(i * 16, 16)] = jnp.full((16,), 1.0, _F32)

    pltpu.sync_copy(zeros1, dbufn)
    for j in range(NPW // RC):
        pltpu.sync_copy(dbufn, degs.at[pl.ds(w * NPW + j * RC, RC)])
    plsc.subcore_barrier()

    base = w * RPW

    def hist_super(i, _):
        pltpu.sync_copy(dst2.at[pl.ds(base + i * SCB, SCB)], ibuf)

        def hist_inner(j, _):
            pltpu.sync_copy(ones128, degs.at[ibuf.at[j]], add=True)
            return 0

        lax.fori_loop(0, SCB, hist_inner, 0)
        return 0

    lax.fori_loop(0, SCH, hist_super, 0)
    plsc.subcore_barrier()

    def do_chunk(r0, rows):
        pltpu.sync_copy(degs.at[pl.ds(r0, rows)], dbufn.at[pl.ds(0, rows)])
        pltpu.sync_copy(maskf.at[pl.ds(r0, rows)], mbuf.at[pl.ds(0, rows)])

        def norm_loop(v, _):
            d = jnp.maximum(dbufn[pl.ds(v * 16, 16)], 1.0)
            nbuf[pl.ds(v * 16, 16)] = _rsqrt16(d)
            return 0

        lax.fori_loop(0, rows // 16, norm_loop, 0)

        def plane_pass(p, _):
            plane = 3 * k + p
            pltpu.sync_copy(labelsS.at[plane].at[pl.ds(r0, rows)],
                            lbuf.at[pl.ds(0, rows)])

            def flat_loop(v, _):
                pos = v * 16 + iota
                r = pos >> 3
                cc = pos - (r << 3)
                nrm = plsc.load_gather(nbuf, [r])
                msk = plsc.load_gather(mbuf, [r])
                lab = plsc.load_gather(lbuf, [r, cc])
                y0 = msk * lab
                plsc.store_scatter(zb, [r, cc], nrm * y0)
                plsc.store_scatter(r2b, [r, cc], 0.1 * nrm * y0)
                plsc.store_scatter(r1b, [r, cc], 0.1 * y0)
                return 0

            lax.fori_loop(0, rows * HQ // 16, flat_loop, 0)
            pltpu.sync_copy(zb.at[pl.ds(0, rows)],
                            zS.at[plane].at[pl.ds(r0, rows)])
            pltpu.sync_copy(r2b.at[pl.ds(0, rows)],
                            R2S.at[plane].at[pl.ds(r0, rows)])
            pltpu.sync_copy(r1b.at[pl.ds(0, rows)],
                            R1S.at[plane].at[pl.ds(r0, rows)])
            return 0

        lax.fori_loop(0, npass, plane_pass, 0)

        @pl.when(k == 0)
        def _():
            def coef_loop(v, _):
                pos = v * 16 + iota
                r = pos >> 3
                cc = pos - (r << 3)
                nrm = plsc.load_gather(nbuf, [r])
                plsc.store_scatter(a2b, [r, cc], 0.9 * nrm * nrm)
                plsc.store_scatter(a1b, [r, cc], 0.9 * nrm)
                return 0

            lax.fori_loop(0, rows * HQ // 16, coef_loop, 0)
            pltpu.sync_copy(a2b.at[pl.ds(0, rows)],
                            A2S.at[pl.ds(r0, rows)])
            pltpu.sync_copy(a1b.at[pl.ds(0, rows)],
                            A1S.at[pl.ds(r0, rows)])

    start = w * NPW
    nfull = jnp.where(w == 15, 16, 17)

    def chunk_body(ci, _):
        do_chunk(start + ci * RC, RC)
        return 0

    lax.fori_loop(0, nfull, chunk_body, 0)

    @pl.when(w == 15)
    def _():
        do_chunk(15 * NPW + 16 * RC, RC_TAIL)


def _main_body(src2, dst2, zS, A2S, R2S, A1S, R1S, zeros2,
               yS, zP, zQ,
               acc, sbuf, dbuf, rbuf, abuf, cbuf1, cbuf2, obuf, zvz):
    w = lax.axis_index("s")
    k = lax.axis_index("c")
    iota = lax.iota(_I32, 16)
    base = w * RPW
    npass = jnp.where(k == 0, 3, 2)
    nch = lax.div(jnp.int32(EW_CHUNKS - 1) - w, jnp.int32(16)) + 1
    pltpu.sync_copy(zeros2, zvz)

    for zin, A, R, zout in ((zS, A2S, R2S, zP),
                            (zP, A2S, R2S, zQ),
                            (zQ, A1S, R1S, yS)):
        def pass_body(p, _, zin=zin, A=A, R=R, zout=zout):
            plane = 3 * k + p
            zk = zin.at[plane]
            rk = R.at[plane]
            ok = zout.at[plane]

            for j in range(NPW // RC):
                pltpu.sync_copy(zvz, acc.at[pl.ds(w * NPW + j * RC, RC)])
            plsc.subcore_barrier()

            def edge_super(i, _):
                pltpu.sync_copy(src2.at[pl.ds(base + i * SCB, SCB)], sbuf)
                pltpu.sync_copy(dst2.at[pl.ds(base + i * SCB, SCB)], dbuf)

                def edge_inner(j, _):
                    pltpu.sync_copy(zk.at[sbuf.at[j]], rbuf)
                    pltpu.sync_copy(rbuf, acc.at[dbuf.at[j]], add=True)
                    return 0

                lax.fori_loop(0, SCB, edge_inner, 0)
                return 0

            lax.fori_loop(0, SCH, edge_super, 0)
            plsc.subcore_barrier()

            def ew_body(i, _):
                r0 = (w + i * 16) * EW_ROWS
                pltpu.sync_copy(acc.at[pl.ds(r0, EW_ROWS)], abuf)
                pltpu.sync_copy(A.at[pl.ds(r0, EW_ROWS)], cbuf1)
                pltpu.sync_copy(rk.at[pl.ds(r0, EW_ROWS)], cbuf2)

                def flat(v, _):
                    pos = v * 16 + iota
                    r = pos >> 3
                    cc = pos - (r << 3)
                    a = plsc.load_gather(abuf, [r, cc])
                    f1 = plsc.load_gather(cbuf1, [r, cc])
                    f2 = plsc.load_gather(cbuf2, [r, cc])
                    plsc.store_scatter(obuf, [r, cc], f1 * a + f2)
                    return 0

                lax.fori_loop(0, EW_ROWS * HQ // 16, flat, 0)
                pltpu.sync_copy(obuf, ok.at[pl.ds(r0, EW_ROWS)])
                return 0

            lax.fori_loop(0, nch, ew_body, 0)
            plsc.subcore_barrier()
            return 0

        lax.fori_loop(0, npass, pass_body, 0)
        plsc.subcore_barrier()


_mesh = plsc.VectorSubcoreMesh(core_axis_name="c", subcore_axis_name="s")

_plane5 = jax.ShapeDtypeStruct((NPL, N, HQ), _F32)
_coef = jax.ShapeDtypeStruct((N, HQ), _F32)

_cparams = pltpu.CompilerParams(needs_layout_passes=False,
                                use_tc_tiling_on_sc=False)

_prep = pl.kernel(
    _prep_body,
    out_type=(_plane5, _plane5, _plane5, _coef, _coef),
    mesh=_mesh,
    compiler_params=_cparams,
    scratch_types=[
        pltpu.VMEM_SHARED((NPAD,), _F32),
        pltpu.VMEM((SCB, B), _I32),
        pltpu.VMEM((B,), _F32),
        pltpu.VMEM((RC,), _F32),
        pltpu.VMEM((RC,), _F32),
        pltpu.VMEM((RC,), _F32),
        pltpu.VMEM((RC, HQ), _F32),
        pltpu.VMEM((RC, HQ), _F32),
        pltpu.VMEM((RC, HQ), _F32),
        pltpu.VMEM((RC, HQ), _F32),
        pltpu.VMEM((RC, HQ), _F32),
        pltpu.VMEM((RC, HQ), _F32),
    ],
)

_main = pl.kernel(
    _main_body,
    out_type=(_plane5,) * 3,
    mesh=_mesh,
    compiler_params=_cparams,
    scratch_types=[
        pltpu.VMEM_SHARED((NACC, HQ), _F32),
        pltpu.VMEM((SCB, B), _I32),
        pltpu.VMEM((SCB, B), _I32),
        pltpu.VMEM((B, HQ), _F32),
        pltpu.VMEM((EW_ROWS, HQ), _F32),
        pltpu.VMEM((EW_ROWS, HQ), _F32),
        pltpu.VMEM((EW_ROWS, HQ), _F32),
        pltpu.VMEM((EW_ROWS, HQ), _F32),
        pltpu.VMEM((RC, HQ), _F32),
    ],
)


def kernel(labels, edge_index, mask):
    npad = EPAD - E
    src_p = jnp.concatenate(
        [edge_index[0], jnp.zeros((npad,), _I32)]).reshape(ROWS2, B)
    dst_p = jnp.concatenate(
        [edge_index[1],
         N + (jnp.arange(npad, dtype=_I32) % 8)]).reshape(ROWS2, B)
    maskf = mask.astype(_F32)
    labelsS = jnp.stack([labels[:, i * HQ:(i + 1) * HQ] for i in range(NPL)])
    zeros1 = jnp.zeros((RC,), _F32)
    zeros2 = jnp.zeros((RC, HQ), _F32)
    zS, R2S, R1S, A2S, A1S = _prep(dst_p, labelsS, maskf, zeros1)
    yS, _, _ = _main(src_p, dst_p, zS, A2S, R2S, A1S, R1S, zeros2)
    return jnp.concatenate([yS[i] for i in range(NPL)], axis=1)


# speedup vs baseline: 4.0752x; 1.4894x over previous
import jax
import jax.numpy as jnp
from jax import lax
from jax.experimental import pallas as pl
from jax.experimental.pallas import tpu as pltpu
from jax.experimental.pallas import tpu_sc as plsc

N = 100000
C = 40
HQ = 8
NPL = 5
E = 1600000
ALPHA = 0.9

B = 128
RPW = 784
ROWS2 = 16 * RPW
EPAD = ROWS2 * B
SCB = 8
SCH = RPW // SCB

NPW = 6256
NPAD = 16 * NPW
RC = 368
RC_TAIL = 272

NACC = NPAD
EW_ROWS = 400
EW_CHUNKS = N // EW_ROWS

_F32 = jnp.float32
_I32 = jnp.int32


def _rsqrt16(d):
    i = lax.bitcast_convert_type(d, _I32)
    i = jnp.int32(0x5F3759DF) - (i >> 1)
    x = lax.bitcast_convert_type(i, _F32)
    for _ in range(3):
        x = x * (1.5 - 0.5 * d * x * x)
    return x


def _prep_body(dst2, labelsS, maskf, zeros1,
               zS, R2S, R1S, A2S, A1S,
               degs, ibuf, ones128, dbufn, mbuf, nbuf,
               lbuf, zb, r2b, r1b, a2b, a1b):
    w = lax.axis_index("s")
    k = lax.axis_index("c")
    iota = lax.iota(_I32, 16)
    npass = jnp.where(k == 0, 3, 2)

    for i in range(8):
        ones128[pl.ds(i * 16, 16)] = jnp.full((16,), 1.0, _F32)

    pltpu.sync_copy(zeros1, dbufn)
    for j in range(NPW // RC):
        pltpu.sync_copy(dbufn, degs.at[pl.ds(w * NPW + j * RC, RC)])
    plsc.subcore_barrier()

    base = w * RPW

    def hist_super(i, _):
        pltpu.sync_copy(dst2.at[pl.ds(base + i * SCB, SCB)], ibuf)

        def hist_inner(j, _):
            pltpu.sync_copy(ones128, degs.at[ibuf.at[j]], add=True)
            return 0

        lax.fori_loop(0, SCB, hist_inner, 0)
        return 0

    lax.fori_loop(0, SCH, hist_super, 0)
    plsc.subcore_barrier()

    def do_chunk(r0, rows):
        pltpu.sync_copy(degs.at[pl.ds(r0, rows)], dbufn.at[pl.ds(0, rows)])
        pltpu.sync_copy(maskf.at[pl.ds(r0, rows)], mbuf.at[pl.ds(0, rows)])

        def norm_loop(v, _):
            d = jnp.maximum(dbufn[pl.ds(v * 16, 16)], 1.0)
            nbuf[pl.ds(v * 16, 16)] = _rsqrt16(d)
            return 0

        lax.fori_loop(0, rows // 16, norm_loop, 0)

        def plane_pass(p, _):
            plane = 3 * k + p
            pltpu.sync_copy(labelsS.at[plane].at[pl.ds(r0, rows)],
                            lbuf.at[pl.ds(0, rows)])

            def flat_loop(v, _):
                pos = v * 16 + iota
                r = pos >> 3
                cc = pos - (r << 3)
                nrm = plsc.load_gather(nbuf, [r])
                msk = plsc.load_gather(mbuf, [r])
                lab = plsc.load_gather(lbuf, [r, cc])
                y0 = msk * lab
                plsc.store_scatter(zb, [r, cc], nrm * y0)
                plsc.store_scatter(r2b, [r, cc], 0.1 * nrm * y0)
                plsc.store_scatter(r1b, [r, cc], 0.1 * y0)
                return 0

            lax.fori_loop(0, rows * HQ // 16, flat_loop, 0)
            pltpu.sync_copy(zb.at[pl.ds(0, rows)],
                            zS.at[plane].at[pl.ds(r0, rows)])
            pltpu.sync_copy(r2b.at[pl.ds(0, rows)],
                            R2S.at[plane].at[pl.ds(r0, rows)])
            pltpu.sync_copy(r1b.at[pl.ds(0, rows)],
                            R1S.at[plane].at[pl.ds(r0, rows)])
            return 0

        lax.fori_loop(0, npass, plane_pass, 0)

        @pl.when(k == 0)
        def _():
            def coef_loop(v, _):
                pos = v * 16 + iota
                r = pos >> 3
                cc = pos - (r << 3)
                nrm = plsc.load_gather(nbuf, [r])
                plsc.store_scatter(a2b, [r, cc], 0.9 * nrm * nrm)
                plsc.store_scatter(a1b, [r, cc], 0.9 * nrm)
                return 0

            lax.fori_loop(0, rows * HQ // 16, coef_loop, 0)
            pltpu.sync_copy(a2b.at[pl.ds(0, rows)],
                            A2S.at[pl.ds(r0, rows)])
            pltpu.sync_copy(a1b.at[pl.ds(0, rows)],
                            A1S.at[pl.ds(r0, rows)])

    start = w * NPW
    nfull = jnp.where(w == 15, 16, 17)

    def chunk_body(ci, _):
        do_chunk(start + ci * RC, RC)
        return 0

    lax.fori_loop(0, nfull, chunk_body, 0)

    @pl.when(w == 15)
    def _():
        do_chunk(15 * NPW + 16 * RC, RC_TAIL)


def _main_body(src2, dst2, zS, A2S, R2S, A1S, R1S, zeros2,
               yS, zP, zQ,
               acc, sbuf, dbuf, rbuf, rbuf2, abuf, cbuf1, cbuf2, obuf, zvz,
               gs0, gs1, ss0, ss1):
    w = lax.axis_index("s")
    k = lax.axis_index("c")
    iota = lax.iota(_I32, 16)
    base = w * RPW
    npass = jnp.where(k == 0, 3, 2)
    nch = lax.div(jnp.int32(EW_CHUNKS - 1) - w, jnp.int32(16)) + 1
    pltpu.sync_copy(zeros2, zvz)

    for zin, A, R, zout in ((zS, A2S, R2S, zP),
                            (zP, A2S, R2S, zQ),
                            (zQ, A1S, R1S, yS)):
        def pass_body(p, _, zin=zin, A=A, R=R, zout=zout):
            plane = 3 * k + p
            zk = zin.at[plane]
            rk = R.at[plane]
            ok = zout.at[plane]

            for j in range(NPW // RC):
                pltpu.sync_copy(zvz, acc.at[pl.ds(w * NPW + j * RC, RC)])
            plsc.subcore_barrier()

            rbs = (rbuf, rbuf2)
            gsem = (gs0, gs1)
            ssem = (ss0, ss1)

            def edge_super(i, _):
                pltpu.sync_copy(src2.at[pl.ds(base + i * SCB, SCB)], sbuf)
                pltpu.sync_copy(dst2.at[pl.ds(base + i * SCB, SCB)], dbuf)
                gds = [None, None]
                sds = [None, None]
                gds[0] = pltpu.async_copy(zk.at[sbuf.at[0]], rbs[0], gsem[0])
                for j in range(SCB):
                    b = j & 1
                    nb = 1 - b
                    if j + 1 < SCB:
                        if sds[nb] is not None:
                            sds[nb].wait()
                        gds[nb] = pltpu.async_copy(
                            zk.at[sbuf.at[j + 1]], rbs[nb], gsem[nb])
                    gds[b].wait()
                    sds[b] = pltpu.async_copy(
                        rbs[b], acc.at[dbuf.at[j]], ssem[b], add=True)
                sds[0].wait()
                sds[1].wait()
                return 0

            lax.fori_loop(0, SCH, edge_super, 0)
            plsc.subcore_barrier()

            def ew_body(i, _):
                r0 = (w + i * 16) * EW_ROWS
                pltpu.sync_copy(acc.at[pl.ds(r0, EW_ROWS)], abuf)
                pltpu.sync_copy(A.at[pl.ds(r0, EW_ROWS)], cbuf1)
                pltpu.sync_copy(rk.at[pl.ds(r0, EW_ROWS)], cbuf2)

                def flat(v, _):
                    pos = v * 16 + iota
                    r = pos >> 3
                    cc = pos - (r << 3)
                    a = plsc.load_gather(abuf, [r, cc])
                    f1 = plsc.load_gather(cbuf1, [r, cc])
                    f2 = plsc.load_gather(cbuf2, [r, cc])
                    plsc.store_scatter(obuf, [r, cc], f1 * a + f2)
                    return 0

                lax.fori_loop(0, EW_ROWS * HQ // 16, flat, 0)
                pltpu.sync_copy(obuf, ok.at[pl.ds(r0, EW_ROWS)])
                return 0

            lax.fori_loop(0, nch, ew_body, 0)
            plsc.subcore_barrier()
            return 0

        lax.fori_loop(0, npass, pass_body, 0)
        plsc.subcore_barrier()


_mesh = plsc.VectorSubcoreMesh(core_axis_name="c", subcore_axis_name="s")

_plane5 = jax.ShapeDtypeStruct((NPL, N, HQ), _F32)
_coef = jax.ShapeDtypeStruct((N, HQ), _F32)

_cparams = pltpu.CompilerParams(needs_layout_passes=False,
                                use_tc_tiling_on_sc=False)

_prep = pl.kernel(
    _prep_body,
    out_type=(_plane5, _plane5, _plane5, _coef, _coef),
    mesh=_mesh,
    compiler_params=_cparams,
    scratch_types=[
        pltpu.VMEM_SHARED((NPAD,), _F32),
        pltpu.VMEM((SCB, B), _I32),
        pltpu.VMEM((B,), _F32),
        pltpu.VMEM((RC,), _F32),
        pltpu.VMEM((RC,), _F32),
        pltpu.VMEM((RC,), _F32),
        pltpu.VMEM((RC, HQ), _F32),
        pltpu.VMEM((RC, HQ), _F32),
        pltpu.VMEM((RC, HQ), _F32),
        pltpu.VMEM((RC, HQ), _F32),
        pltpu.VMEM((RC, HQ), _F32),
        pltpu.VMEM((RC, HQ), _F32),
    ],
)

_main = pl.kernel(
    _main_body,
    out_type=(_plane5,) * 3,
    mesh=_mesh,
    compiler_params=_cparams,
    scratch_types=[
        pltpu.VMEM_SHARED((NACC, HQ), _F32),
        pltpu.VMEM((SCB, B), _I32),
        pltpu.VMEM((SCB, B), _I32),
        pltpu.VMEM((B, HQ), _F32),
        pltpu.VMEM((B, HQ), _F32),
        pltpu.VMEM((EW_ROWS, HQ), _F32),
        pltpu.VMEM((EW_ROWS, HQ), _F32),
        pltpu.VMEM((EW_ROWS, HQ), _F32),
        pltpu.VMEM((EW_ROWS, HQ), _F32),
        pltpu.VMEM((RC, HQ), _F32),
        pltpu.SemaphoreType.DMA,
        pltpu.SemaphoreType.DMA,
        pltpu.SemaphoreType.DMA,
        pltpu.SemaphoreType.DMA,
    ],
)


def kernel(labels, edge_index, mask):
    npad = EPAD - E
    src_p = jnp.concatenate(
        [edge_index[0], jnp.zeros((npad,), _I32)]).reshape(ROWS2, B)
    dst_p = jnp.concatenate(
        [edge_index[1],
         N + (jnp.arange(npad, dtype=_I32) % 8)]).reshape(ROWS2, B)
    maskf = mask.astype(_F32)
    labelsS = jnp.stack([labels[:, i * HQ:(i + 1) * HQ] for i in range(NPL)])
    zeros1 = jnp.zeros((RC,), _F32)
    zeros2 = jnp.zeros((RC, HQ), _F32)
    zS, R2S, R1S, A2S, A1S = _prep(dst_p, labelsS, maskf, zeros1)
    yS, _, _ = _main(src_p, dst_p, zS, A2S, R2S, A1S, R1S, zeros2)
    return jnp.concatenate([yS[i] for i in range(NPL)], axis=1)


# speedup vs baseline: 5.8835x; 1.4438x over previous
import jax
import jax.numpy as jnp
from jax import lax
from jax.experimental import pallas as pl
from jax.experimental.pallas import tpu as pltpu
from jax.experimental.pallas import tpu_sc as plsc

N = 100000
C = 40
HQ = 8
NPL = 5
E = 1600000
ALPHA = 0.9

B = 128
RPW = 784
ROWS2 = 16 * RPW
EPAD = ROWS2 * B
SCB = 16
SCH = RPW // SCB
DBUF = 4

NPW = 6256
NPAD = 16 * NPW
RC = 368
RC_TAIL = 272

NACC = NPAD
EW_ROWS = 400
EW_CHUNKS = N // EW_ROWS

_F32 = jnp.float32
_I32 = jnp.int32


def _rsqrt16(d):
    i = lax.bitcast_convert_type(d, _I32)
    i = jnp.int32(0x5F3759DF) - (i >> 1)
    x = lax.bitcast_convert_type(i, _F32)
    for _ in range(3):
        x = x * (1.5 - 0.5 * d * x * x)
    return x


def _prep_body(dst2, labelsS, maskf, zeros1,
               zS, R2S, R1S, A2S, A1S,
               degs, ibuf, ones128, dbufn, mbuf, nbuf,
               lbuf, zb, r2b, r1b, a2b, a1b):
    w = lax.axis_index("s")
    k = lax.axis_index("c")
    iota = lax.iota(_I32, 16)
    npass = jnp.where(k == 0, 3, 2)

    for i in range(8):
        ones128[pl.ds(i * 16, 16)] = jnp.full((16,), 1.0, _F32)

    pltpu.sync_copy(zeros1, dbufn)
    for j in range(NPW // RC):
        pltpu.sync_copy(dbufn, degs.at[pl.ds(w * NPW + j * RC, RC)])
    plsc.subcore_barrier()

    base = w * RPW

    def hist_super(i, _):
        pltpu.sync_copy(dst2.at[pl.ds(base + i * SCB, SCB)], ibuf)

        def hist_inner(j, _):
            pltpu.sync_copy(ones128, degs.at[ibuf.at[j]], add=True)
            return 0

        lax.fori_loop(0, SCB, hist_inner, 0)
        return 0

    lax.fori_loop(0, SCH, hist_super, 0)
    plsc.subcore_barrier()

    def do_chunk(r0, rows):
        pltpu.sync_copy(degs.at[pl.ds(r0, rows)], dbufn.at[pl.ds(0, rows)])
        pltpu.sync_copy(maskf.at[pl.ds(r0, rows)], mbuf.at[pl.ds(0, rows)])

        def norm_loop(v, _):
            d = jnp.maximum(dbufn[pl.ds(v * 16, 16)], 1.0)
            nbuf[pl.ds(v * 16, 16)] = _rsqrt16(d)
            return 0

        lax.fori_loop(0, rows // 16, norm_loop, 0)

        def plane_pass(p, _):
            plane = 3 * k + p
            pltpu.sync_copy(labelsS.at[plane].at[pl.ds(r0, rows)],
                            lbuf.at[pl.ds(0, rows)])

            def flat_loop(v, _):
                pos = v * 16 + iota
                r = pos >> 3
                cc = pos - (r << 3)
                nrm = plsc.load_gather(nbuf, [r])
                msk = plsc.load_gather(mbuf, [r])
                lab = plsc.load_gather(lbuf, [r, cc])
                y0 = msk * lab
                plsc.store_scatter(zb, [r, cc], nrm * y0)
                plsc.store_scatter(r2b, [r, cc], 0.1 * nrm * y0)
                plsc.store_scatter(r1b, [r, cc], 0.1 * y0)
                return 0

            lax.fori_loop(0, rows * HQ // 16, flat_loop, 0)
            pltpu.sync_copy(zb.at[pl.ds(0, rows)],
                            zS.at[plane].at[pl.ds(r0, rows)])
            pltpu.sync_copy(r2b.at[pl.ds(0, rows)],
                            R2S.at[plane].at[pl.ds(r0, rows)])
            pltpu.sync_copy(r1b.at[pl.ds(0, rows)],
                            R1S.at[plane].at[pl.ds(r0, rows)])
            return 0

        lax.fori_loop(0, npass, plane_pass, 0)

        @pl.when(k == 0)
        def _():
            def coef_loop(v, _):
                pos = v * 16 + iota
                r = pos >> 3
                cc = pos - (r << 3)
                nrm = plsc.load_gather(nbuf, [r])
                plsc.store_scatter(a2b, [r, cc], 0.9 * nrm * nrm)
                plsc.store_scatter(a1b, [r, cc], 0.9 * nrm)
                return 0

            lax.fori_loop(0, rows * HQ // 16, coef_loop, 0)
            pltpu.sync_copy(a2b.at[pl.ds(0, rows)],
                            A2S.at[pl.ds(r0, rows)])
            pltpu.sync_copy(a1b.at[pl.ds(0, rows)],
                            A1S.at[pl.ds(r0, rows)])

    start = w * NPW
    nfull = jnp.where(w == 15, 16, 17)

    def chunk_body(ci, _):
        do_chunk(start + ci * RC, RC)
        return 0

    lax.fori_loop(0, nfull, chunk_body, 0)

    @pl.when(w == 15)
    def _():
        do_chunk(15 * NPW + 16 * RC, RC_TAIL)


def _main_body(src2, dst2, zS, A2S, R2S, A1S, R1S, zeros2,
               yS, zP, zQ,
               acc, sbuf, dbuf, rb0, rb1, rb2, rb3,
               abuf, cbuf1, cbuf2, obuf, zvz,
               gs0, gs1, gs2, gs3, ss0, ss1, ss2, ss3):
    w = lax.axis_index("s")
    k = lax.axis_index("c")
    iota = lax.iota(_I32, 16)
    base = w * RPW
    npass = jnp.where(k == 0, 3, 2)
    nch = lax.div(jnp.int32(EW_CHUNKS - 1) - w, jnp.int32(16)) + 1
    pltpu.sync_copy(zeros2, zvz)

    for zin, A, R, zout in ((zS, A2S, R2S, zP),
                            (zP, A2S, R2S, zQ),
                            (zQ, A1S, R1S, yS)):
        def pass_body(p, _, zin=zin, A=A, R=R, zout=zout):
            plane = 3 * k + p
            zk = zin.at[plane]
            rk = R.at[plane]
            ok = zout.at[plane]

            for j in range(NPW // RC):
                pltpu.sync_copy(zvz, acc.at[pl.ds(w * NPW + j * RC, RC)])
            plsc.subcore_barrier()

            rbs = (rb0, rb1, rb2, rb3)
            gsem = (gs0, gs1, gs2, gs3)
            ssem = (ss0, ss1, ss2, ss3)

            def edge_super(i, _):
                pltpu.sync_copy(src2.at[pl.ds(base + i * SCB, SCB)], sbuf)
                pltpu.sync_copy(dst2.at[pl.ds(base + i * SCB, SCB)], dbuf)
                gds = [None] * DBUF
                sds = [None] * DBUF
                for m in range(DBUF - 1):
                    gds[m] = pltpu.async_copy(
                        zk.at[sbuf.at[m]], rbs[m], gsem[m])
                for j in range(SCB):
                    b = j % DBUF
                    m = j + DBUF - 1
                    if m < SCB:
                        mb = m % DBUF
                        if sds[mb] is not None:
                            sds[mb].wait()
                        gds[mb] = pltpu.async_copy(
                            zk.at[sbuf.at[m]], rbs[mb], gsem[mb])
                    gds[b].wait()
                    sds[b] = pltpu.async_copy(
                        rbs[b], acc.at[dbuf.at[j]], ssem[b], add=True)
                for d in range(DBUF):
                    if sds[d] is not None:
                        sds[d].wait()
                return 0

            lax.fori_loop(0, SCH, edge_super, 0)
            plsc.subcore_barrier()

            def ew_body(i, _):
                r0 = (w + i * 16) * EW_ROWS
                pltpu.sync_copy(acc.at[pl.ds(r0, EW_ROWS)], abuf)
                pltpu.sync_copy(A.at[pl.ds(r0, EW_ROWS)], cbuf1)
                pltpu.sync_copy(rk.at[pl.ds(r0, EW_ROWS)], cbuf2)

                def flat(v, _):
                    pos = v * 16 + iota
                    r = pos >> 3
                    cc = pos - (r << 3)
                    a = plsc.load_gather(abuf, [r, cc])
                    f1 = plsc.load_gather(cbuf1, [r, cc])
                    f2 = plsc.load_gather(cbuf2, [r, cc])
                    plsc.store_scatter(obuf, [r, cc], f1 * a + f2)
                    return 0

                lax.fori_loop(0, EW_ROWS * HQ // 16, flat, 0)
                pltpu.sync_copy(obuf, ok.at[pl.ds(r0, EW_ROWS)])
                return 0

            lax.fori_loop(0, nch, ew_body, 0)
            plsc.subcore_barrier()
            return 0

        lax.fori_loop(0, npass, pass_body, 0)
        plsc.subcore_barrier()


_mesh = plsc.VectorSubcoreMesh(core_axis_name="c", subcore_axis_name="s")

_plane5 = jax.ShapeDtypeStruct((NPL, N, HQ), _F32)
_coef = jax.ShapeDtypeStruct((N, HQ), _F32)

_cparams = pltpu.CompilerParams(needs_layout_passes=False,
                                use_tc_tiling_on_sc=False)

_prep = pl.kernel(
    _prep_body,
    out_type=(_plane5, _plane5, _plane5, _coef, _coef),
    mesh=_mesh,
    compiler_params=_cparams,
    scratch_types=[
        pltpu.VMEM_SHARED((NPAD,), _F32),
        pltpu.VMEM((SCB, B), _I32),
        pltpu.VMEM((B,), _F32),
        pltpu.VMEM((RC,), _F32),
        pltpu.VMEM((RC,), _F32),
        pltpu.VMEM((RC,), _F32),
        pltpu.VMEM((RC, HQ), _F32),
        pltpu.VMEM((RC, HQ), _F32),
        pltpu.VMEM((RC, HQ), _F32),
        pltpu.VMEM((RC, HQ), _F32),
        pltpu.VMEM((RC, HQ), _F32),
        pltpu.VMEM((RC, HQ), _F32),
    ],
)

_main = pl.kernel(
    _main_body,
    out_type=(_plane5,) * 3,
    mesh=_mesh,
    compiler_params=_cparams,
    scratch_types=[
        pltpu.VMEM_SHARED((NACC, HQ), _F32),
        pltpu.VMEM((SCB, B), _I32),
        pltpu.VMEM((SCB, B), _I32),
        pltpu.VMEM((B, HQ), _F32),
        pltpu.VMEM((B, HQ), _F32),
        pltpu.VMEM((B, HQ), _F32),
        pltpu.VMEM((B, HQ), _F32),
        pltpu.VMEM((EW_ROWS, HQ), _F32),
        pltpu.VMEM((EW_ROWS, HQ), _F32),
        pltpu.VMEM((EW_ROWS, HQ), _F32),
        pltpu.VMEM((EW_ROWS, HQ), _F32),
        pltpu.VMEM((RC, HQ), _F32),
        pltpu.SemaphoreType.DMA,
        pltpu.SemaphoreType.DMA,
        pltpu.SemaphoreType.DMA,
        pltpu.SemaphoreType.DMA,
        pltpu.SemaphoreType.DMA,
        pltpu.SemaphoreType.DMA,
        pltpu.SemaphoreType.DMA,
        pltpu.SemaphoreType.DMA,
    ],
)


def kernel(labels, edge_index, mask):
    npad = EPAD - E
    src_p = jnp.concatenate(
        [edge_index[0], jnp.zeros((npad,), _I32)]).reshape(ROWS2, B)
    dst_p = jnp.concatenate(
        [edge_index[1],
         N + (jnp.arange(npad, dtype=_I32) % 8)]).reshape(ROWS2, B)
    maskf = mask.astype(_F32)
    labelsS = jnp.stack([labels[:, i * HQ:(i + 1) * HQ] for i in range(NPL)])
    zeros1 = jnp.zeros((RC,), _F32)
    zeros2 = jnp.zeros((RC, HQ), _F32)
    zS, R2S, R1S, A2S, A1S = _prep(dst_p, labelsS, maskf, zeros1)
    yS, _, _ = _main(src_p, dst_p, zS, A2S, R2S, A1S, R1S, zeros2)
    return jnp.concatenate([yS[i] for i in range(NPL)], axis=1)


# speedup vs baseline: 6.7659x; 1.1500x over previous
import jax
import jax.numpy as jnp
from jax import lax
from jax.experimental import pallas as pl
from jax.experimental.pallas import tpu as pltpu
from jax.experimental.pallas import tpu_sc as plsc

N = 100000
C = 40
HQ = 8
NPL = 5
E = 1600000
ALPHA = 0.9

B = 128
RPW = 784
ROWS2 = 16 * RPW
EPAD = ROWS2 * B
SCB = 16
SCH = RPW // SCB
DBUF = 8

NPW = 6256
NPAD = 16 * NPW
RC = 368
RC_TAIL = 272

NACC = NPAD
EW_ROWS = 400
EW_CHUNKS = N // EW_ROWS

_F32 = jnp.float32
_I32 = jnp.int32


def _rsqrt16(d):
    i = lax.bitcast_convert_type(d, _I32)
    i = jnp.int32(0x5F3759DF) - (i >> 1)
    x = lax.bitcast_convert_type(i, _F32)
    for _ in range(3):
        x = x * (1.5 - 0.5 * d * x * x)
    return x


def _prep_body(dst2, labelsS, maskf, zeros1,
               zS, R2S, R1S, A2S, A1S,
               degs, ibuf, ones128, dbufn, mbuf, nbuf,
               lbuf, zb, r2b, r1b, a2b, a1b):
    w = lax.axis_index("s")
    k = lax.axis_index("c")
    iota = lax.iota(_I32, 16)
    npass = jnp.where(k == 0, 3, 2)

    for i in range(8):
        ones128[pl.ds(i * 16, 16)] = jnp.full((16,), 1.0, _F32)

    pltpu.sync_copy(zeros1, dbufn)
    for j in range(NPW // RC):
        pltpu.sync_copy(dbufn, degs.at[pl.ds(w * NPW + j * RC, RC)])
    plsc.subcore_barrier()

    base = w * RPW

    def hist_super(i, _):
        pltpu.sync_copy(dst2.at[pl.ds(base + i * SCB, SCB)], ibuf)

        def hist_inner(j, _):
            pltpu.sync_copy(ones128, degs.at[ibuf.at[j]], add=True)
            return 0

        lax.fori_loop(0, SCB, hist_inner, 0)
        return 0

    lax.fori_loop(0, SCH, hist_super, 0)
    plsc.subcore_barrier()

    def do_chunk(r0, rows):
        pltpu.sync_copy(degs.at[pl.ds(r0, rows)], dbufn.at[pl.ds(0, rows)])
        pltpu.sync_copy(maskf.at[pl.ds(r0, rows)], mbuf.at[pl.ds(0, rows)])

        def norm_loop(v, _):
            d = jnp.maximum(dbufn[pl.ds(v * 16, 16)], 1.0)
            nbuf[pl.ds(v * 16, 16)] = _rsqrt16(d)
            return 0

        lax.fori_loop(0, rows // 16, norm_loop, 0)

        def plane_pass(p, _):
            plane = 3 * k + p
            pltpu.sync_copy(labelsS.at[plane].at[pl.ds(r0, rows)],
                            lbuf.at[pl.ds(0, rows)])

            def flat_loop(v, _):
                pos = v * 16 + iota
                r = pos >> 3
                cc = pos - (r << 3)
                nrm = plsc.load_gather(nbuf, [r])
                msk = plsc.load_gather(mbuf, [r])
                lab = plsc.load_gather(lbuf, [r, cc])
                y0 = msk * lab
                plsc.store_scatter(zb, [r, cc], nrm * y0)
                plsc.store_scatter(r2b, [r, cc], 0.1 * nrm * y0)
                plsc.store_scatter(r1b, [r, cc], 0.1 * y0)
                return 0

            lax.fori_loop(0, rows * HQ // 16, flat_loop, 0)
            pltpu.sync_copy(zb.at[pl.ds(0, rows)],
                            zS.at[plane].at[pl.ds(r0, rows)])
            pltpu.sync_copy(r2b.at[pl.ds(0, rows)],
                            R2S.at[plane].at[pl.ds(r0, rows)])
            pltpu.sync_copy(r1b.at[pl.ds(0, rows)],
                            R1S.at[plane].at[pl.ds(r0, rows)])
            return 0

        lax.fori_loop(0, npass, plane_pass, 0)

        @pl.when(k == 0)
        def _():
            def coef_loop(v, _):
                pos = v * 16 + iota
                r = pos >> 3
                cc = pos - (r << 3)
                nrm = plsc.load_gather(nbuf, [r])
                plsc.store_scatter(a2b, [r, cc], 0.9 * nrm * nrm)
                plsc.store_scatter(a1b, [r, cc], 0.9 * nrm)
                return 0

            lax.fori_loop(0, rows * HQ // 16, coef_loop, 0)
            pltpu.sync_copy(a2b.at[pl.ds(0, rows)],
                            A2S.at[pl.ds(r0, rows)])
            pltpu.sync_copy(a1b.at[pl.ds(0, rows)],
                            A1S.at[pl.ds(r0, rows)])

    start = w * NPW
    nfull = jnp.where(w == 15, 16, 17)

    def chunk_body(ci, _):
        do_chunk(start + ci * RC, RC)
        return 0

    lax.fori_loop(0, nfull, chunk_body, 0)

    @pl.when(w == 15)
    def _():
        do_chunk(15 * NPW + 16 * RC, RC_TAIL)


def _main_body(src2, dst2, zS, A2S, R2S, A1S, R1S, zeros2,
               yS, zP, zQ,
               acc, sbuf, dbuf, rb0, rb1, rb2, rb3, rb4, rb5, rb6, rb7,
               abuf, cbuf1, cbuf2, obuf, zvz,
               gs0, gs1, gs2, gs3, gs4, gs5, gs6, gs7,
               ss0, ss1, ss2, ss3, ss4, ss5, ss6, ss7):
    w = lax.axis_index("s")
    k = lax.axis_index("c")
    iota = lax.iota(_I32, 16)
    base = w * RPW
    npass = jnp.where(k == 0, 3, 2)
    nch = lax.div(jnp.int32(EW_CHUNKS - 1) - w, jnp.int32(16)) + 1
    pltpu.sync_copy(zeros2, zvz)

    for zin, A, R, zout in ((zS, A2S, R2S, zP),
                            (zP, A2S, R2S, zQ),
                            (zQ, A1S, R1S, yS)):
        def pass_body(p, _, zin=zin, A=A, R=R, zout=zout):
            plane = 3 * k + p
            zk = zin.at[plane]
            rk = R.at[plane]
            ok = zout.at[plane]

            for j in range(NPW // RC):
                pltpu.sync_copy(zvz, acc.at[pl.ds(w * NPW + j * RC, RC)])
            plsc.subcore_barrier()

            rbs = (rb0, rb1, rb2, rb3, rb4, rb5, rb6, rb7)
            gsem = (gs0, gs1, gs2, gs3, gs4, gs5, gs6, gs7)
            ssem = (ss0, ss1, ss2, ss3, ss4, ss5, ss6, ss7)

            def edge_super(i, _):
                pltpu.sync_copy(src2.at[pl.ds(base + i * SCB, SCB)], sbuf)
                pltpu.sync_copy(dst2.at[pl.ds(base + i * SCB, SCB)], dbuf)
                gds = [None] * DBUF
                sds = [None] * DBUF
                for m in range(DBUF - 1):
                    gds[m] = pltpu.async_copy(
                        zk.at[sbuf.at[m]], rbs[m], gsem[m])
                for j in range(SCB):
                    b = j % DBUF
                    m = j + DBUF - 1
                    if m < SCB:
                        mb = m % DBUF
                        if sds[mb] is not None:
                            sds[mb].wait()
                        gds[mb] = pltpu.async_copy(
                            zk.at[sbuf.at[m]], rbs[mb], gsem[mb])
                    gds[b].wait()
                    sds[b] = pltpu.async_copy(
                        rbs[b], acc.at[dbuf.at[j]], ssem[b], add=True)
                for d in range(DBUF):
                    if sds[d] is not None:
                        sds[d].wait()
                return 0

            lax.fori_loop(0, SCH, edge_super, 0)
            plsc.subcore_barrier()

            def ew_body(i, _):
                r0 = (w + i * 16) * EW_ROWS
                pltpu.sync_copy(acc.at[pl.ds(r0, EW_ROWS)], abuf)
                pltpu.sync_copy(A.at[pl.ds(r0, EW_ROWS)], cbuf1)
                pltpu.sync_copy(rk.at[pl.ds(r0, EW_ROWS)], cbuf2)

                def flat(v, _):
                    pos = v * 16 + iota
                    r = pos >> 3
                    cc = pos - (r << 3)
                    a = plsc.load_gather(abuf, [r, cc])
                    f1 = plsc.load_gather(cbuf1, [r, cc])
                    f2 = plsc.load_gather(cbuf2, [r, cc])
                    plsc.store_scatter(obuf, [r, cc], f1 * a + f2)
                    return 0

                lax.fori_loop(0, EW_ROWS * HQ // 16, flat, 0)
                pltpu.sync_copy(obuf, ok.at[pl.ds(r0, EW_ROWS)])
                return 0

            lax.fori_loop(0, nch, ew_body, 0)
            plsc.subcore_barrier()
            return 0

        lax.fori_loop(0, npass, pass_body, 0)
        plsc.subcore_barrier()


_mesh = plsc.VectorSubcoreMesh(core_axis_name="c", subcore_axis_name="s")

_plane5 = jax.ShapeDtypeStruct((NPL, N, HQ), _F32)
_coef = jax.ShapeDtypeStruct((N, HQ), _F32)

_cparams = pltpu.CompilerParams(needs_layout_passes=False,
                                use_tc_tiling_on_sc=False)

_prep = pl.kernel(
    _prep_body,
    out_type=(_plane5, _plane5, _plane5, _coef, _coef),
    mesh=_mesh,
    compiler_params=_cparams,
    scratch_types=[
        pltpu.VMEM_SHARED((NPAD,), _F32),
        pltpu.VMEM((SCB, B), _I32),
        pltpu.VMEM((B,), _F32),
        pltpu.VMEM((RC,), _F32),
        pltpu.VMEM((RC,), _F32),
        pltpu.VMEM((RC,), _F32),
        pltpu.VMEM((RC, HQ), _F32),
        pltpu.VMEM((RC, HQ), _F32),
        pltpu.VMEM((RC, HQ), _F32),
        pltpu.VMEM((RC, HQ), _F32),
        pltpu.VMEM((RC, HQ), _F32),
        pltpu.VMEM((RC, HQ), _F32),
    ],
)

_main = pl.kernel(
    _main_body,
    out_type=(_plane5,) * 3,
    mesh=_mesh,
    compiler_params=_cparams,
    scratch_types=[
        pltpu.VMEM_SHARED((NACC, HQ), _F32),
        pltpu.VMEM((SCB, B), _I32),
        pltpu.VMEM((SCB, B), _I32),
        pltpu.VMEM((B, HQ), _F32),
        pltpu.VMEM((B, HQ), _F32),
        pltpu.VMEM((B, HQ), _F32),
        pltpu.VMEM((B, HQ), _F32),
        pltpu.VMEM((B, HQ), _F32),
        pltpu.VMEM((B, HQ), _F32),
        pltpu.VMEM((B, HQ), _F32),
        pltpu.VMEM((B, HQ), _F32),
        pltpu.VMEM((EW_ROWS, HQ), _F32),
        pltpu.VMEM((EW_ROWS, HQ), _F32),
        pltpu.VMEM((EW_ROWS, HQ), _F32),
        pltpu.VMEM((EW_ROWS, HQ), _F32),
        pltpu.VMEM((RC, HQ), _F32),
        pltpu.SemaphoreType.DMA,
        pltpu.SemaphoreType.DMA,
        pltpu.SemaphoreType.DMA,
        pltpu.SemaphoreType.DMA,
        pltpu.SemaphoreType.DMA,
        pltpu.SemaphoreType.DMA,
        pltpu.SemaphoreType.DMA,
        pltpu.SemaphoreType.DMA,
        pltpu.SemaphoreType.DMA,
        pltpu.SemaphoreType.DMA,
        pltpu.SemaphoreType.DMA,
        pltpu.SemaphoreType.DMA,
        pltpu.SemaphoreType.DMA,
        pltpu.SemaphoreType.DMA,
        pltpu.SemaphoreType.DMA,
        pltpu.SemaphoreType.DMA,
    ],
)


def kernel(labels, edge_index, mask):
    npad = EPAD - E
    src_p = jnp.concatenate(
        [edge_index[0], jnp.zeros((npad,), _I32)]).reshape(ROWS2, B)
    dst_p = jnp.concatenate(
        [edge_index[1],
         N + (jnp.arange(npad, dtype=_I32) % 8)]).reshape(ROWS2, B)
    maskf = mask.astype(_F32)
    labelsS = jnp.stack([labels[:, i * HQ:(i + 1) * HQ] for i in range(NPL)])
    zeros1 = jnp.zeros((RC,), _F32)
    zeros2 = jnp.zeros((RC, HQ), _F32)
    zS, R2S, R1S, A2S, A1S = _prep(dst_p, labelsS, maskf, zeros1)
    yS, _, _ = _main(src_p, dst_p, zS, A2S, R2S, A1S, R1S, zeros2)
    return jnp.concatenate([yS[i] for i in range(NPL)], axis=1)
